# baseline (speedup 1.0000x reference)
import functools

import jax
import jax.numpy as jnp
from jax import lax
from jax.experimental import pallas as pl
from jax.experimental.pallas import tpu as pltpu
from jax.experimental.pallas import tpu_sc as plsc

_DIM = 64
_NW = 32
_CHUNK = 100
_NBUF = 8


@functools.lru_cache(maxsize=None)
def _make_gather(n_rows: int, n_cols: int):
    mesh = plsc.VectorSubcoreMesh(core_axis_name="c", subcore_axis_name="s")
    n_chunk = (n_rows // _NW) * (n_cols // _CHUNK)
    rows_per_w = n_rows // _NW
    cpr = n_cols // _CHUNK
    n_grp = n_chunk // _NBUF

    @functools.partial(
        pl.kernel,
        out_type=jax.ShapeDtypeStruct((n_rows, n_cols, _DIM), jnp.float32),
        mesh=mesh,
        compiler_params=pltpu.CompilerParams(use_tc_tiling_on_sc=False),
        scratch_types=[pltpu.VMEM((n_chunk, _CHUNK), jnp.int32)]
        + [pltpu.VMEM((1, _CHUNK, _DIM), jnp.float32)] * _NBUF
        + [pltpu.SemaphoreType.DMA] * (2 * _NBUF),
    )
    def k(idx_hbm, table_hbm, out_hbm, idx_v, *rest):
        rows = rest[:_NBUF]
        gsem = rest[_NBUF:2 * _NBUF]
        osem = rest[2 * _NBUF:]
        wid = lax.axis_index("s") * 2 + lax.axis_index("c")
        base_i = wid * rows_per_w
        pltpu.sync_copy(idx_hbm.at[wid], idx_v)

        def out_slice(b, c):
            i = base_i + c // cpr
            j0 = (b % cpr) * _CHUNK
            return out_hbm.at[pl.ds(i, 1), pl.ds(j0, _CHUNK)]

        def fire_gather(b, c):
            pltpu.async_copy(table_hbm.at[idx_v.at[c]], rows[b].at[0], gsem[b])

        def wait_gather(b, c):
            pltpu.make_async_copy(table_hbm.at[idx_v.at[c]], rows[b].at[0], gsem[b]).wait()

        def fire_out(b, c):
            pltpu.async_copy(rows[b], out_slice(b, c), osem[b])

        def wait_out(b, c):
            pltpu.make_async_copy(rows[b], out_slice(b, c), osem[b]).wait()

        for b in range(_NBUF):
            fire_gather(b, b)

        @pl.loop(0, n_grp - 1)
        def _(g):
            c0 = g * _NBUF
            for b in range(_NBUF):
                wait_gather(b, c0 + b)
                fire_out(b, c0 + b)
            for b in range(_NBUF):
                wait_out(b, c0 + b)
                fire_gather(b, c0 + _NBUF + b)

        c0 = (n_grp - 1) * _NBUF
        for b in range(_NBUF):
            wait_gather(b, c0 + b)
            fire_out(b, c0 + b)
        for b in range(_NBUF):
            wait_out(b, c0 + b)

    return k


def kernel(emb_indices, table):
    n_rows, n_cols = emb_indices.shape
    idx = emb_indices.reshape(_NW, -1, _CHUNK)
    return _make_gather(n_rows, n_cols)(idx, table)
